# Initial kernel scaffold; baseline (speedup 1.0000x reference)
import jax
import jax.numpy as jnp
from jax.experimental import pallas as pl


def kernel(adj, x, W, gamma, beta):
    raise NotImplementedError("write your pallas kernel here")


# speedup vs baseline: 1.0874x; 1.0874x over previous
import jax
import jax.numpy as jnp
from jax.experimental import pallas as pl
from jax.experimental.pallas import tpu as pltpu

_N = 10000
_D = 128
_BLK = 400
_EPS = 1e-5


def _gcn_kernel(adj_ref, x_ref, w_ref, gamma_ref, beta_ref, out_ref,
                z_ref, s_ref, ss_ref):
    i = pl.program_id(0)

    @pl.when(i == 0)
    def _init():
        z_ref[...] = jnp.dot(x_ref[...], w_ref[...],
                             preferred_element_type=jnp.float32)
        s_ref[...] = jnp.zeros_like(s_ref)
        ss_ref[...] = jnp.zeros_like(ss_ref)

    y = jnp.dot(adj_ref[...], z_ref[...], preferred_element_type=jnp.float32)
    out_ref[pl.ds(i * _BLK, _BLK), :] = y
    s_ref[...] += jnp.sum(y, axis=0, keepdims=True)
    ss_ref[...] += jnp.sum(y * y, axis=0, keepdims=True)

    @pl.when(i == pl.num_programs(0) - 1)
    def _finish():
        mean = s_ref[...] / _N
        var = ss_ref[...] / _N - mean * mean
        scale = gamma_ref[...] * jax.lax.rsqrt(var + _EPS)
        shift = beta_ref[...] - mean * scale
        out_ref[...] = jnp.maximum(out_ref[...] * scale + shift, 0.0)


@jax.jit
def _gcn(adj, x, W, gamma, beta):
    return pl.pallas_call(
        _gcn_kernel,
        grid=(_N // _BLK,),
        in_specs=[
            pl.BlockSpec((_BLK, _N), lambda i: (i, 0)),
            pl.BlockSpec((_N, _D), lambda i: (0, 0)),
            pl.BlockSpec((_D, _D), lambda i: (0, 0)),
            pl.BlockSpec((1, _D), lambda i: (0, 0)),
            pl.BlockSpec((1, _D), lambda i: (0, 0)),
        ],
        out_specs=pl.BlockSpec((_N, _D), lambda i: (0, 0)),
        out_shape=jax.ShapeDtypeStruct((_N, _D), jnp.float32),
        scratch_shapes=[
            pltpu.VMEM((_N, _D), jnp.float32),
            pltpu.VMEM((1, _D), jnp.float32),
            pltpu.VMEM((1, _D), jnp.float32),
        ],
        compiler_params=pltpu.CompilerParams(
            dimension_semantics=("arbitrary",),
        ),
    )(adj, x, W, gamma.reshape(1, _D), beta.reshape(1, _D))


def kernel(adj, x, W, gamma, beta):
    return _gcn(adj, x, W, gamma, beta)
